# baseline (device time: 12816 ns/iter reference)
import jax
import jax.numpy as jnp
from jax import lax
from jax.experimental import pallas as pl
from jax.experimental.pallas import tpu as pltpu


def kernel(x, dest):
    m_per, n = x.shape
    m_glob = 2 * m_per
    dest2d = dest.reshape(1, m_per)

    def body(x_ref, dest_ref, out_ref, xg, destg, send_sems, recv_sems):
        my_x = lax.axis_index("x")
        my_y = lax.axis_index("y")
        nbr = (my_x, 1 - my_y)

        barrier_sem = pltpu.get_barrier_semaphore()
        pl.semaphore_signal(
            barrier_sem, inc=1, device_id=nbr,
            device_id_type=pl.DeviceIdType.MESH,
        )
        pl.semaphore_wait(barrier_sem, 1)

        xg[pl.ds(my_y * m_per, m_per), :] = x_ref[...]
        destg[pl.ds(my_y, 1), :] = dest_ref[...]

        rdma_x = pltpu.make_async_remote_copy(
            src_ref=x_ref,
            dst_ref=xg.at[pl.ds(my_y * m_per, m_per)],
            send_sem=send_sems.at[0],
            recv_sem=recv_sems.at[0],
            device_id=nbr,
            device_id_type=pl.DeviceIdType.MESH,
        )
        rdma_d = pltpu.make_async_remote_copy(
            src_ref=dest_ref,
            dst_ref=destg.at[pl.ds(my_y, 1)],
            send_sem=send_sems.at[1],
            recv_sem=recv_sems.at[1],
            device_id=nbr,
            device_id_type=pl.DeviceIdType.MESH,
        )
        rdma_x.start()
        rdma_d.start()

        row_i = lax.broadcasted_iota(jnp.int32, (m_glob, m_glob), 0)
        col_i = lax.broadcasted_iota(jnp.int32, (m_glob, m_glob), 1)
        tri = (row_i < col_i).astype(jnp.float32)
        r_iota = lax.broadcasted_iota(jnp.int32, (m_per, m_glob), 0)

        rdma_d.wait()

        d_f = destg[...].astype(jnp.float32)
        y_f = my_y.astype(jnp.float32)
        diff = d_f - y_f
        m_f32 = (1.0 - diff * diff).reshape(1, m_glob)
        ex = jnp.dot(m_f32, tri, preferred_element_type=jnp.float32)
        exi = ex.astype(jnp.int32)
        onehot = jnp.where(exi == r_iota, 1.0, 0.0)
        sel = onehot * m_f32

        rdma_x.wait()
        out_ref[...] = jnp.dot(sel, xg[...], preferred_element_type=jnp.float32)

    return pl.pallas_call(
        body,
        out_shape=jax.ShapeDtypeStruct((m_per, n), jnp.float32),
        in_specs=[
            pl.BlockSpec(memory_space=pltpu.VMEM),
            pl.BlockSpec(memory_space=pltpu.VMEM),
        ],
        out_specs=pl.BlockSpec(memory_space=pltpu.VMEM),
        scratch_shapes=[
            pltpu.VMEM((m_glob, n), jnp.float32),
            pltpu.VMEM((2, m_per), jnp.int32),
            pltpu.SemaphoreType.DMA((2,)),
            pltpu.SemaphoreType.DMA((2,)),
        ],
        compiler_params=pltpu.CompilerParams(collective_id=0),
    )(x, dest2d)


# device time: 10174 ns/iter; 1.2597x vs baseline; 1.2597x over previous
import jax
import jax.numpy as jnp
from jax import lax
from jax.experimental import pallas as pl
from jax.experimental.pallas import tpu as pltpu

CHUNK = 64
NCHUNK = 8


def kernel(x, dest):
    m_per, n = x.shape
    assert CHUNK * NCHUNK == m_per
    dest2d = dest.reshape(1, m_per)

    def body(x_ref, dest_ref, out_ref, sendbuf, recvbuf, cnt_tx, cnt_rx,
             sems_small, csend_sems, crecv_sems):
        my_x = lax.axis_index("x")
        my_y = lax.axis_index("y")
        nbr = (my_x, 1 - my_y)
        y_f = my_y.astype(jnp.float32)

        d_loc = dest_ref[...].astype(jnp.float32)
        diff = d_loc - y_f
        mask_send = diff * diff
        mask_keep = 1.0 - mask_send
        cnt_send = jnp.sum(mask_send)
        cnt_keep = float(m_per) - cnt_send

        row_i = lax.broadcasted_iota(jnp.int32, (m_per, m_per), 0)
        col_i = lax.broadcasted_iota(jnp.int32, (m_per, m_per), 1)
        tri = (row_i < col_i).astype(jnp.float32)
        row_f = row_i.astype(jnp.float32)
        col_f = col_i.astype(jnp.float32)

        ex_send = jnp.dot(mask_send, tri, preferred_element_type=jnp.float32)
        sel_send = jnp.where(ex_send - row_f == 0.0, 1.0, 0.0) * mask_send
        sendbuf[...] = jnp.dot(
            sel_send, x_ref[...], preferred_element_type=jnp.float32
        )
        ex_keep = jnp.dot(mask_keep, tri, preferred_element_type=jnp.float32)

        cnt_tx[...] = jnp.zeros((1, 128), jnp.float32) + cnt_send

        barrier_sem = pltpu.get_barrier_semaphore()
        pl.semaphore_signal(
            barrier_sem, inc=1, device_id=nbr,
            device_id_type=pl.DeviceIdType.MESH,
        )
        pl.semaphore_wait(barrier_sem, 1)

        rdma_cnt = pltpu.make_async_remote_copy(
            src_ref=cnt_tx,
            dst_ref=cnt_rx,
            send_sem=sems_small.at[0],
            recv_sem=sems_small.at[1],
            device_id=nbr,
            device_id_type=pl.DeviceIdType.MESH,
        )
        rdma_cnt.start()

        chunk_rdmas = []
        for c in range(NCHUNK):
            r = pltpu.make_async_remote_copy(
                src_ref=sendbuf.at[pl.ds(c * CHUNK, CHUNK)],
                dst_ref=recvbuf.at[pl.ds(c * CHUNK, CHUNK)],
                send_sem=csend_sems.at[c],
                recv_sem=crecv_sems.at[c],
                device_id=nbr,
                device_id_type=pl.DeviceIdType.MESH,
            )
            chunk_rdmas.append(r)

            @pl.when(cnt_send > float(c * CHUNK))
            def _():
                r.start()

        rdma_cnt.wait()
        cnt_recv = cnt_rx[0, 0]
        off_k = y_f * cnt_recv
        off_r = (1.0 - y_f) * cnt_keep

        p_a = jnp.where(ex_keep + off_k - row_f == 0.0, 1.0, 0.0) * mask_keep
        p_b = jnp.where(col_f + off_r - row_f == 0.0, 1.0, 0.0) * jnp.where(
            col_f - cnt_recv < 0.0, 1.0, 0.0
        )

        for c in range(NCHUNK):
            @pl.when(cnt_send > float(c * CHUNK))
            def _():
                chunk_rdmas[c].wait_send()

            @pl.when(cnt_recv > float(c * CHUNK))
            def _():
                chunk_rdmas[c].wait_recv()

        out_ref[...] = jnp.dot(
            p_a, x_ref[...], preferred_element_type=jnp.float32
        ) + jnp.dot(p_b, recvbuf[...], preferred_element_type=jnp.float32)

    return pl.pallas_call(
        body,
        out_shape=jax.ShapeDtypeStruct((m_per, n), jnp.float32),
        in_specs=[
            pl.BlockSpec(memory_space=pltpu.VMEM),
            pl.BlockSpec(memory_space=pltpu.VMEM),
        ],
        out_specs=pl.BlockSpec(memory_space=pltpu.VMEM),
        scratch_shapes=[
            pltpu.VMEM((m_per, n), jnp.float32),
            pltpu.VMEM((m_per, n), jnp.float32),
            pltpu.VMEM((1, 128), jnp.float32),
            pltpu.VMEM((1, 128), jnp.float32),
            pltpu.SemaphoreType.DMA((2,)),
            pltpu.SemaphoreType.DMA((NCHUNK,)),
            pltpu.SemaphoreType.DMA((NCHUNK,)),
        ],
        compiler_params=pltpu.CompilerParams(collective_id=0),
    )(x, dest2d)
